# Initial kernel scaffold; baseline (speedup 1.0000x reference)
import jax
import jax.numpy as jnp
from jax.experimental import pallas as pl


def kernel(x, edge_index, edge_attr, W1, b1, W2, b2, gamma, beta):
    raise NotImplementedError("write your pallas kernel here")


# speedup vs baseline: 5.4814x; 5.4814x over previous
import functools

import jax
import jax.numpy as jnp
from jax import lax
from jax.experimental import pallas as pl
from jax.experimental.pallas import tpu as pltpu
from jax.experimental.pallas import tpu_sc as plsc

N_NODES = 10000
D = 128
N_EDGES = 320000
BN_EPS = 1e-5

NC = 2
NS = 16
NW = NC * NS
EDGES_PER_TILE = N_EDGES // NW
CHUNK = 80
NITER = EDGES_PER_TILE // CHUNK
N_PAD = 10240
ROWS_PER_TILE = N_PAD // NS


def _sc_segment_sum(x, src, dst, zeros_tile):
    mesh = plsc.VectorSubcoreMesh(core_axis_name="c", subcore_axis_name="s")

    @functools.partial(
        pl.kernel,
        out_type=jax.ShapeDtypeStruct((NC, N_PAD, D), jnp.float32),
        mesh=mesh,
        scratch_types=[
            pltpu.VMEM((CHUNK,), jnp.int32),
            pltpu.VMEM((CHUNK,), jnp.int32),
            pltpu.VMEM((CHUNK, D), jnp.float32),
            pltpu.VMEM_SHARED((N_PAD, D), jnp.float32),
            pltpu.SemaphoreType.DMA,
        ],
    )
    def seg_sum(x_hbm, src_hbm, dst_hbm, zero_hbm, out_hbm,
                src_v, dst_v, rows_v, agg_sh, sem):
        c = lax.axis_index("c")
        s = lax.axis_index("s")
        wid = s * NC + c
        base = wid * EDGES_PER_TILE

        pltpu.sync_copy(zero_hbm, agg_sh.at[pl.ds(s * ROWS_PER_TILE, ROWS_PER_TILE)])
        plsc.subcore_barrier()

        def body(j, carry):
            off = base + j * CHUNK
            pltpu.sync_copy(src_hbm.at[pl.ds(off, CHUNK)], src_v)
            pltpu.sync_copy(dst_hbm.at[pl.ds(off, CHUNK)], dst_v)
            pltpu.async_copy(x_hbm.at[src_v], rows_v, sem).wait()
            pltpu.sync_copy(rows_v, agg_sh.at[dst_v], add=True)
            return carry

        lax.fori_loop(0, NITER, body, 0)
        plsc.subcore_barrier()

        r0 = s * ROWS_PER_TILE
        pltpu.sync_copy(agg_sh.at[pl.ds(r0, ROWS_PER_TILE)],
                        out_hbm.at[c].at[pl.ds(r0, ROWS_PER_TILE)])

    return seg_sum(x, src, dst, zeros_tile)


def _tc_mlp_bn(x, partials, W1, b1, W2, b2, gamma, beta):
    def body(x_ref, p_ref, w1_ref, b1_ref, w2_ref, b2_ref, g_ref, bt_ref, o_ref):
        h = x_ref[...] + p_ref[0, :N_NODES] + p_ref[1, :N_NODES]
        h = jnp.dot(h, w1_ref[...], preferred_element_type=jnp.float32) + b1_ref[...]
        h = jnp.maximum(h, 0.0)
        h = jnp.dot(h, w2_ref[...], preferred_element_type=jnp.float32) + b2_ref[...]
        h = jnp.maximum(h, 0.0)
        mean = jnp.sum(h, axis=0, keepdims=True) * (1.0 / N_NODES)
        d0 = h - mean
        var = jnp.sum(d0 * d0, axis=0, keepdims=True) * (1.0 / N_NODES)
        inv = lax.rsqrt(var + BN_EPS)
        o_ref[...] = g_ref[...] * d0 * inv + bt_ref[...]

    return pl.pallas_call(
        body,
        out_shape=jax.ShapeDtypeStruct((N_NODES, D), jnp.float32),
    )(x, partials, W1, b1, W2, b2, gamma, beta)


@jax.jit
def kernel(x, edge_index, edge_attr, W1, b1, W2, b2, gamma, beta):
    src = edge_index[0].astype(jnp.int32)
    dst = edge_index[1].astype(jnp.int32)
    zeros_tile = jnp.zeros((ROWS_PER_TILE, D), jnp.float32)
    partials = _sc_segment_sum(x, src, dst, zeros_tile)
    return _tc_mlp_bn(x, partials,
                      W1, b1.reshape(1, D), W2, b2.reshape(1, D),
                      gamma.reshape(1, D), beta.reshape(1, D))


# speedup vs baseline: 11.7232x; 2.1387x over previous
import functools

import jax
import jax.numpy as jnp
from jax import lax
from jax.experimental import pallas as pl
from jax.experimental.pallas import tpu as pltpu
from jax.experimental.pallas import tpu_sc as plsc

N_NODES = 10000
D = 128
N_EDGES = 320000
BN_EPS = 1e-5

NC = 2
NS = 16
NW = NC * NS
EDGES_PER_TILE = N_EDGES // NW
CHUNK = 80
NITER = EDGES_PER_TILE // CHUNK
N_PAD = 10240
ROWS_PER_TILE = N_PAD // NS


NBUF = 2


def _sc_segment_sum(x, packed3, zeros_tile):
    mesh = plsc.VectorSubcoreMesh(core_axis_name="c", subcore_axis_name="s")

    @functools.partial(
        pl.kernel,
        out_type=jax.ShapeDtypeStruct((NC, N_PAD, D), jnp.float32),
        mesh=mesh,
        scratch_types=[
            pltpu.VMEM((NITER, CHUNK), jnp.int32),
            pltpu.VMEM((NBUF, CHUNK), jnp.int32),
            pltpu.VMEM((NBUF, CHUNK), jnp.int32),
            pltpu.VMEM((NBUF, CHUNK, D), jnp.float32),
            pltpu.VMEM_SHARED((N_PAD, D), jnp.float32),
            pltpu.SemaphoreType.DMA((NBUF,)),
        ],
    )
    def seg_sum(x_hbm, pk_hbm, zero_hbm, out_hbm,
                pk_v, su_v, du_v, rows_v, agg_sh, gsem):
        c = lax.axis_index("c")
        s = lax.axis_index("s")
        wid = s * NC + c

        pltpu.sync_copy(pk_hbm.at[wid], pk_v)
        pltpu.sync_copy(zero_hbm, agg_sh.at[pl.ds(s * ROWS_PER_TILE, ROWS_PER_TILE)])
        plsc.subcore_barrier()

        def gather(j, k):
            for t in range(CHUNK // 16):
                w = pk_v[j, pl.ds(t * 16, 16)]
                su_v[k, pl.ds(t * 16, 16)] = lax.bitwise_and(w, 0xFFFF)
                du_v[k, pl.ds(t * 16, 16)] = lax.shift_right_logical(w, 16)
            pltpu.async_copy(x_hbm.at[su_v.at[k]], rows_v.at[k], gsem.at[k])

        def consume(j, k):
            pltpu.make_async_copy(x_hbm.at[su_v.at[k]], rows_v.at[k],
                                  gsem.at[k]).wait()
            pltpu.sync_copy(rows_v.at[k], agg_sh.at[du_v.at[k]], add=True)

        for k in range(NBUF):
            gather(k, k)

        def body(i, carry):
            for k in range(NBUF):
                j = NBUF * i + k
                consume(j, k)

                @pl.when(j + NBUF < NITER)
                def _(j=j, k=k):
                    gather(j + NBUF, k)
            return carry

        lax.fori_loop(0, (NITER - 1) // NBUF, body, 0)
        for j in range(NBUF * ((NITER - 1) // NBUF), NITER):
            consume(j, j % NBUF)

        plsc.subcore_barrier()
        r0 = s * ROWS_PER_TILE
        pltpu.sync_copy(agg_sh.at[pl.ds(r0, ROWS_PER_TILE)],
                        out_hbm.at[c].at[pl.ds(r0, ROWS_PER_TILE)])

    return seg_sum(x, packed3, zeros_tile)


def _tc_mlp_bn(x, partials, W1, b1, W2, b2, gamma, beta):
    def body(x_ref, p_ref, w1_ref, b1_ref, w2_ref, b2_ref, g_ref, bt_ref, o_ref):
        h = x_ref[...] + p_ref[0, :N_NODES] + p_ref[1, :N_NODES]
        h = jnp.dot(h, w1_ref[...], preferred_element_type=jnp.float32) + b1_ref[...]
        h = jnp.maximum(h, 0.0)
        h = jnp.dot(h, w2_ref[...], preferred_element_type=jnp.float32) + b2_ref[...]
        h = jnp.maximum(h, 0.0)
        mean = jnp.sum(h, axis=0, keepdims=True) * (1.0 / N_NODES)
        d0 = h - mean
        var = jnp.sum(d0 * d0, axis=0, keepdims=True) * (1.0 / N_NODES)
        inv = lax.rsqrt(var + BN_EPS)
        o_ref[...] = g_ref[...] * d0 * inv + bt_ref[...]

    return pl.pallas_call(
        body,
        out_shape=jax.ShapeDtypeStruct((N_NODES, D), jnp.float32),
    )(x, partials, W1, b1, W2, b2, gamma, beta)


@jax.jit
def kernel(x, edge_index, edge_attr, W1, b1, W2, b2, gamma, beta):
    src = edge_index[0].astype(jnp.int32)
    dst = edge_index[1].astype(jnp.int32)
    packed3 = (src | (dst << 16)).reshape(NW, NITER, CHUNK)
    zeros_tile = jnp.zeros((ROWS_PER_TILE, D), jnp.float32)
    partials = _sc_segment_sum(x, packed3, zeros_tile)
    return _tc_mlp_bn(x, partials,
                      W1, b1.reshape(1, D), W2, b2.reshape(1, D),
                      gamma.reshape(1, D), beta.reshape(1, D))


# speedup vs baseline: 11.7607x; 1.0032x over previous
import functools

import jax
import jax.numpy as jnp
from jax import lax
from jax.experimental import pallas as pl
from jax.experimental.pallas import tpu as pltpu
from jax.experimental.pallas import tpu_sc as plsc

N_NODES = 10000
D = 128
N_EDGES = 320000
BN_EPS = 1e-5

NC = 2
NS = 16
NW = NC * NS
EDGES_PER_TILE = N_EDGES // NW
CHUNK = 80
NITER = EDGES_PER_TILE // CHUNK
N_PAD = 10240
ROWS_PER_TILE = N_PAD // NS


NBUF = 2


def _sc_segment_sum(x, packed3, zeros_tile):
    mesh = plsc.VectorSubcoreMesh(core_axis_name="c", subcore_axis_name="s")

    @functools.partial(
        pl.kernel,
        out_type=jax.ShapeDtypeStruct((NC, N_PAD, D), jnp.float32),
        mesh=mesh,
        scratch_types=[
            pltpu.VMEM((NITER, CHUNK), jnp.int32),
            pltpu.VMEM((NBUF, CHUNK), jnp.int32),
            pltpu.VMEM((NBUF, CHUNK), jnp.int32),
            pltpu.VMEM((NBUF, CHUNK, D), jnp.float32),
            pltpu.VMEM_SHARED((N_PAD, D), jnp.float32),
            pltpu.SemaphoreType.DMA((NBUF,)),
            pltpu.SemaphoreType.DMA((NBUF,)),
        ],
    )
    def seg_sum(x_hbm, pk_hbm, zero_hbm, out_hbm,
                pk_v, su_v, du_v, rows_v, agg_sh, gsem, ssem):
        c = lax.axis_index("c")
        s = lax.axis_index("s")
        wid = s * NC + c

        pltpu.sync_copy(pk_hbm.at[wid], pk_v)
        pltpu.sync_copy(zero_hbm, agg_sh.at[pl.ds(s * ROWS_PER_TILE, ROWS_PER_TILE)])
        plsc.subcore_barrier()

        def gather(j, k):
            for t in range(CHUNK // 16):
                w = pk_v[j, pl.ds(t * 16, 16)]
                su_v[k, pl.ds(t * 16, 16)] = lax.bitwise_and(w, 0xFFFF)
                du_v[k, pl.ds(t * 16, 16)] = lax.shift_right_logical(w, 16)
            pltpu.async_copy(x_hbm.at[su_v.at[k]], rows_v.at[k], gsem.at[k])

        def gather_wait(k):
            pltpu.make_async_copy(x_hbm.at[su_v.at[k]], rows_v.at[k],
                                  gsem.at[k]).wait()

        def scatter(k):
            pltpu.async_copy(rows_v.at[k], agg_sh.at[du_v.at[k]], ssem.at[k],
                             add=True)

        def scatter_wait(k):
            pltpu.make_async_copy(rows_v.at[k], agg_sh.at[du_v.at[k]],
                                  ssem.at[k]).wait()

        gather(0, 0)

        def step(j, k):
            @pl.when(j >= 1)
            def _():
                scatter_wait(1 - k)

            @pl.when(j + 1 < NITER)
            def _():
                gather(j + 1, 1 - k)

            gather_wait(k)
            scatter(k)

        def body(i, carry):
            for k in range(NBUF):
                j = NBUF * i + k

                @pl.when(j < NITER)
                def _(j=j, k=k):
                    step(j, k)
            return carry

        lax.fori_loop(0, (NITER + NBUF - 1) // NBUF, body, 0)
        scatter_wait((NITER - 1) % NBUF)
        plsc.subcore_barrier()
        r0 = s * ROWS_PER_TILE
        pltpu.sync_copy(agg_sh.at[pl.ds(r0, ROWS_PER_TILE)],
                        out_hbm.at[c].at[pl.ds(r0, ROWS_PER_TILE)])

    return seg_sum(x, packed3, zeros_tile)


def _tc_mlp_bn(x, partials, W1, b1, W2, b2, gamma, beta):
    def body(x_ref, p_ref, w1_ref, b1_ref, w2_ref, b2_ref, g_ref, bt_ref, o_ref):
        h = x_ref[...] + p_ref[0, :N_NODES] + p_ref[1, :N_NODES]
        h = jnp.dot(h, w1_ref[...], preferred_element_type=jnp.float32) + b1_ref[...]
        h = jnp.maximum(h, 0.0)
        h = jnp.dot(h, w2_ref[...], preferred_element_type=jnp.float32) + b2_ref[...]
        h = jnp.maximum(h, 0.0)
        mean = jnp.sum(h, axis=0, keepdims=True) * (1.0 / N_NODES)
        d0 = h - mean
        var = jnp.sum(d0 * d0, axis=0, keepdims=True) * (1.0 / N_NODES)
        inv = lax.rsqrt(var + BN_EPS)
        o_ref[...] = g_ref[...] * d0 * inv + bt_ref[...]

    return pl.pallas_call(
        body,
        out_shape=jax.ShapeDtypeStruct((N_NODES, D), jnp.float32),
    )(x, partials, W1, b1, W2, b2, gamma, beta)


@jax.jit
def kernel(x, edge_index, edge_attr, W1, b1, W2, b2, gamma, beta):
    src = edge_index[0].astype(jnp.int32)
    dst = edge_index[1].astype(jnp.int32)
    packed3 = (src | (dst << 16)).reshape(NW, NITER, CHUNK)
    zeros_tile = jnp.zeros((ROWS_PER_TILE, D), jnp.float32)
    partials = _sc_segment_sum(x, packed3, zeros_tile)
    return _tc_mlp_bn(x, partials,
                      W1, b1.reshape(1, D), W2, b2.reshape(1, D),
                      gamma.reshape(1, D), beta.reshape(1, D))
